# baseline (speedup 1.0000x reference)
import functools

import jax
import jax.numpy as jnp
from jax import lax
from jax.experimental import pallas as pl
from jax.experimental.pallas import tpu as pltpu

N = 10000
E = 320000
D = 128
B = 64
NUM_CLASSES = 10
K_POOL = 50
NUM_ITER = 3
HID = 32
OD = NUM_CLASSES * 32


def _routing_body(priors_ref, out_ref):
    priors = priors_ref[...]
    out = jnp.mean(priors, axis=0)
    lane = lax.broadcasted_iota(jnp.int32, (OD, NUM_CLASSES), 0) // 32
    cls = lax.broadcasted_iota(jnp.int32, (OD, NUM_CLASSES), 1)
    S = (lane == cls).astype(jnp.float32)

    def block_sum(v):
        return jax.lax.dot(v, S, precision=lax.Precision.HIGHEST)

    def expand(v):
        return jax.lax.dot(v, S.T, precision=lax.Precision.HIGHEST)

    for _ in range(NUM_ITER):
        nrm = jnp.sqrt(block_sum(out * out))
        no = out * expand(1.0 / (nrm + 1e-12))
        logits = jnp.einsum('ibd,dc->ibc', priors * no[None], S,
                            precision=lax.Precision.HIGHEST)
        m = jnp.max(logits, axis=0)
        p = jnp.exp(logits - m[None])
        denom = jnp.sum(p, axis=0)
        probs = p / denom[None]
        pe = jnp.einsum('ibc,dc->ibd', probs, S,
                        precision=lax.Precision.HIGHEST)
        out = jnp.sum(pe * priors, axis=0)

    sq = block_sum(out * out)
    scale = (sq / (1.0 + sq)) / jnp.sqrt(sq + 1e-12)
    out = out * expand(scale)
    cls_norm = jnp.sqrt(block_sum(out * out))
    out_ref[...] = cls_norm


def _routing(priors_ibod):
    return pl.pallas_call(
        _routing_body,
        out_shape=jax.ShapeDtypeStruct((B, NUM_CLASSES), jnp.float32),
    )(priors_ibod)


def kernel(x, edge_index, batch, W1, b1, W2, b2, W3, b3, Wc):
    src0, dst0 = edge_index[0], edge_index[1]
    keep = (src0 != dst0).astype(jnp.float32)
    loop = jnp.arange(N, dtype=src0.dtype)
    src = jnp.concatenate([src0, loop])
    dst = jnp.concatenate([dst0, loop])
    ew = jnp.concatenate([keep, jnp.ones((N,), jnp.float32)])

    deg = jnp.zeros((N,), jnp.float32).at[dst].add(ew)
    dinv = jnp.where(deg > 0, 1.0 / jnp.sqrt(jnp.maximum(deg, 1e-12)), 0.0)
    coef = ew * dinv[src] * dinv[dst]

    def gcn(h):
        msg = h[src] * coef[:, None]
        return jnp.zeros((N, h.shape[1]), h.dtype).at[dst].add(msg)

    x1 = jnp.tanh(gcn(x @ W1) + b1)
    x2 = jnp.tanh(gcn(x1 @ W2) + b2)
    x3 = jnp.tanh(gcn(x2 @ W3) + b3)
    xc = jnp.concatenate([x1, x2, x3], axis=-1)

    scores = jnp.where(batch[None, :] == jnp.arange(B)[:, None],
                       xc[:, -1][None, :], -jnp.inf)
    vals, idx = jax.lax.top_k(scores, K_POOL)
    pooled = xc[idx]
    pooled = jnp.where(jnp.isfinite(vals)[..., None], pooled, 0.0)

    Wt = jnp.transpose(Wc, (1, 3, 0, 2)).reshape(K_POOL, 96, OD)
    priors = jnp.einsum('bic,icd->ibd', pooled, Wt,
                        precision=lax.Precision.HIGHEST)
    return _routing(priors)


# speedup vs baseline: 7.8689x; 7.8689x over previous
import functools

import jax
import jax.numpy as jnp
from jax import lax
from jax.experimental import pallas as pl
from jax.experimental.pallas import tpu as pltpu
from jax.experimental.pallas import tpu_sc as plsc

N = 10000
NP = 10240
E = 320000
D = 128
B = 64
NUM_CLASSES = 10
K_POOL = 50
NUM_ITER = 3
HID = 32
OD = NUM_CLASSES * 32
DUMMY = 10016

NTILE = 32
ECH = 79
EPT = ECH * 128
EPAD = NTILE * EPT
BCH = 3
STRIPE = NP // 16

ROWS = 320
WIN = 8192
NWIN = 40
EFLAT = NWIN * WIN
CAP = 16384

_MESH = plsc.VectorSubcoreMesh(core_axis_name="c", subcore_axis_name="s",
                               num_cores=2, num_subcores=16)
_SC_PARAMS = pltpu.CompilerParams(use_tc_tiling_on_sc=False,
                                  needs_layout_passes=False)


def _fill2d(ref, rows, val):
    def body(r, _):
        ref[r, pl.ds(0, 16)] = jnp.full((16,), val, ref.dtype)
        if ref.shape[1] == 32:
            ref[r, pl.ds(16, 16)] = jnp.full((16,), val, ref.dtype)
        return 0
    lax.fori_loop(0, rows, body, 0)


def _fill1d(ref, n, val):
    def body(i, _):
        ref[pl.ds(i * 16, 16)] = jnp.full((16,), val, ref.dtype)
        return 0
    lax.fori_loop(0, n // 16, body, 0)


def _edgeprep_body(src_hbm, dst_hbm, batch_hbm, srcf_hbm, dstf_hbm,
                   degp_hbm, cntp_hbm, sl_hbm, sd_hbm, cnts_hbm,
                   srcv, dstv, bv, ones, zb, sw, dw, listS, listD, ct,
                   acc, cacc):
    c = lax.axis_index("c")
    s = lax.axis_index("s")
    w = c * 16 + s
    pltpu.sync_copy(src_hbm.at[w], srcv)
    pltpu.sync_copy(dst_hbm.at[w], dstv)
    pltpu.sync_copy(batch_hbm.at[w], bv)
    _fill2d(ones, 128, 1.0)
    _fill2d(zb, 128, 0.0)
    for i in range(STRIPE // 128):
        pltpu.sync_copy(zb, acc.at[pl.ds(s * STRIPE + i * 128, 128)])
    @pl.when(s == 0)
    def _():
        pltpu.sync_copy(zb, cacc)
    plsc.subcore_barrier()

    def fix(i, _):
        j = i // 8
        t = (i % 8) * 16
        vs = srcv[j, pl.ds(t, 16)]
        vd = dstv[j, pl.ds(t, 16)]
        dstv[j, pl.ds(t, 16)] = jnp.where(vs == vd, DUMMY, vd)
        return 0
    lax.fori_loop(0, ECH * 8, fix, 0)

    def scat(j, _):
        pltpu.sync_copy(ones, acc.at[dstv.at[j]], add=True)
        return 0
    lax.fori_loop(0, ECH, scat, 0)
    for j in range(BCH):
        pltpu.sync_copy(ones, cacc.at[bv.at[j]], add=True)

    _fill1d(listS, CAP, 0)
    _fill1d(listD, CAP, DUMMY)
    r0 = w * ROWS
    iota = lax.iota(jnp.int32, 16)

    def win(jw, cnt):
        pltpu.sync_copy(srcf_hbm.at[pl.ds(jw * WIN, WIN)], sw)
        pltpu.sync_copy(dstf_hbm.at[pl.ds(jw * WIN, WIN)], dw)

        def chunk(k, cnt2):
            vs = sw[pl.ds(k * 16, 16)]
            vd = dw[pl.ds(k * 16, 16)]
            mask = (vd >= r0) & (vd < r0 + ROWS) & (vs != vd)
            rank = plsc.cumsum(jnp.where(mask, 1, 0))
            tot = jnp.max(rank)
            @pl.when(tot > 0)
            def _():
                pos = cnt2 + rank - 1
                plsc.store_scatter(listS, [pos], vs, mask=mask)
                plsc.store_scatter(listD, [pos], vd, mask=mask)
            return cnt2 + tot
        return lax.fori_loop(0, WIN // 16, chunk, cnt)
    cnt = lax.fori_loop(0, NWIN, win, 0)

    pltpu.sync_copy(listS, sl_hbm.at[w])
    pltpu.sync_copy(listD, sd_hbm.at[w])
    ct[...] = jnp.where(iota == 0, cnt, 0)
    pltpu.sync_copy(ct, cnts_hbm.at[w])

    plsc.subcore_barrier()
    pltpu.sync_copy(acc.at[pl.ds(s * STRIPE, STRIPE)],
                    degp_hbm.at[c, pl.ds(s * STRIPE, STRIPE)])
    @pl.when(s == 0)
    def _():
        pltpu.sync_copy(cacc, cntp_hbm.at[c])


_edgeprep = functools.partial(
    pl.kernel, mesh=_MESH, compiler_params=_SC_PARAMS,
    out_type=(jax.ShapeDtypeStruct((2, NP, 16), jnp.float32),
              jax.ShapeDtypeStruct((2, 128, 16), jnp.float32),
              jax.ShapeDtypeStruct((NTILE, CAP), jnp.int32),
              jax.ShapeDtypeStruct((NTILE, CAP), jnp.int32),
              jax.ShapeDtypeStruct((NTILE, 16), jnp.int32)),
    scratch_types=[pltpu.VMEM((ECH, 128), jnp.int32),
                   pltpu.VMEM((ECH, 128), jnp.int32),
                   pltpu.VMEM((BCH, 128), jnp.int32),
                   pltpu.VMEM((128, 16), jnp.float32),
                   pltpu.VMEM((128, 16), jnp.float32),
                   pltpu.VMEM((WIN,), jnp.int32),
                   pltpu.VMEM((WIN,), jnp.int32),
                   pltpu.VMEM((CAP,), jnp.int32),
                   pltpu.VMEM((CAP,), jnp.int32),
                   pltpu.VMEM((16,), jnp.int32),
                   pltpu.VMEM_SHARED((NP, 16), jnp.float32),
                   pltpu.VMEM_SHARED((128, 16), jnp.float32)],
)(_edgeprep_body)


def _apply_body(h_hbm, sl_hbm, sd_hbm, cnts_hbm, dinv_hbm, p_hbm,
                lS, lD, dv, hb, g0, g1, ct, acc, sem0, sem1):
    c = lax.axis_index("c")
    s = lax.axis_index("s")
    w = c * 16 + s
    r0 = w * ROWS
    pltpu.sync_copy(sl_hbm.at[w], lS)
    pltpu.sync_copy(sd_hbm.at[w], lD)
    pltpu.sync_copy(dinv_hbm, dv)
    pltpu.sync_copy(h_hbm.at[pl.ds(r0, ROWS)], hb)
    pltpu.sync_copy(cnts_hbm.at[w], ct)
    _fill2d(acc, ROWS + 16, 0.0)
    cnt = ct[...][0]
    T = (cnt + 127) // 128
    iota = lax.iota(jnp.int32, 16)
    bufs = (g0, g1)
    sems = (sem0, sem1)

    def start(j, b):
        pltpu.async_copy(h_hbm.at[lS.at[pl.ds(j * 128, 128)]], bufs[b],
                         sems[b])

    def wait(b):
        pltpu.make_async_copy(h_hbm.at[pl.ds(0, 128)], bufs[b],
                              sems[b]).wait()

    def process(j, b):
        buf = bufs[b]

        def group(g, _):
            base = j * 128 + g * 16
            sv = lS[pl.ds(base, 16)]
            vd = lD[pl.ds(base, 16)]
            coef = plsc.load_gather(dv, [sv]) * plsc.load_gather(dv, [vd])
            inr = (vd >= r0) & (vd < r0 + ROWS)
            dloc = jnp.where(inr, vd - r0, ROWS)
            for u in range(16):
                c_u = jnp.sum(jnp.where(iota == u, coef, 0.0))
                dl_u = jnp.sum(jnp.where(iota == u, dloc, 0))
                r = g * 16 + u
                plsc.addupdate(acc.at[dl_u, pl.ds(0, 16)],
                               buf[r, pl.ds(0, 16)] * c_u)
                plsc.addupdate(acc.at[dl_u, pl.ds(16, 16)],
                               buf[r, pl.ds(16, 16)] * c_u)
            return 0
        lax.fori_loop(0, 8, group, 0)

    @pl.when(T > 0)
    def _():
        start(0, 0)

    def outer(jj, _):
        for b in range(2):
            j = jj * 2 + b
            @pl.when(j < T)
            def _():
                wait(b)
                @pl.when(j + 1 < T)
                def _():
                    start(j + 1, 1 - b)
                process(j, b)
        return 0
    lax.fori_loop(0, (T + 1) // 2, outer, 0)

    def selfrow(i, _):
        d16 = dv[pl.ds(r0 + i * 16, 16)]
        coef = d16 * d16
        for u in range(16):
            r = i * 16 + u
            n = r0 + r
            c_u = jnp.sum(jnp.where(iota == u, coef, 0.0))
            @pl.when(n < N)
            def _():
                plsc.addupdate(acc.at[r, pl.ds(0, 16)],
                               hb[r, pl.ds(0, 16)] * c_u)
                plsc.addupdate(acc.at[r, pl.ds(16, 16)],
                               hb[r, pl.ds(16, 16)] * c_u)
        return 0
    lax.fori_loop(0, ROWS // 16, selfrow, 0)

    pltpu.sync_copy(acc.at[pl.ds(0, ROWS)], p_hbm.at[pl.ds(r0, ROWS)])


_sc_apply = functools.partial(
    pl.kernel, mesh=_MESH, compiler_params=_SC_PARAMS,
    out_type=jax.ShapeDtypeStruct((NP, HID), jnp.float32),
    scratch_types=[pltpu.VMEM((CAP,), jnp.int32),
                   pltpu.VMEM((CAP,), jnp.int32),
                   pltpu.VMEM((NP,), jnp.float32),
                   pltpu.VMEM((ROWS, HID), jnp.float32),
                   pltpu.VMEM((128, HID), jnp.float32),
                   pltpu.VMEM((128, HID), jnp.float32),
                   pltpu.VMEM((16,), jnp.int32),
                   pltpu.VMEM((ROWS + 16, HID), jnp.float32),
                   pltpu.SemaphoreType.DMA,
                   pltpu.SemaphoreType.DMA],
)(_apply_body)


_GB = 8
_RB = NP // _GB


def _first_body(x_ref, w_ref, h_ref):
    h_ref[...] = jax.lax.dot(x_ref[...], w_ref[...])


def _tc_first(x_pad, W1):
    return pl.pallas_call(
        _first_body,
        grid=(_GB,),
        in_specs=[pl.BlockSpec((_RB, D), lambda g: (g, 0)),
                  pl.BlockSpec((D, HID), lambda g: (0, 0))],
        out_specs=pl.BlockSpec((_RB, HID), lambda g: (g, 0)),
        out_shape=jax.ShapeDtypeStruct((NP, HID), jnp.float32),
    )(x_pad, W1)


def _layer_body(p_ref, b_ref, w_ref, x_ref, hn_ref):
    a = jnp.tanh(p_ref[...] + b_ref[...])
    x_ref[...] = a
    hn_ref[...] = jax.lax.dot(a, w_ref[...])


def _tc_layer(p, bvec, Wn):
    return pl.pallas_call(
        _layer_body,
        grid=(_GB,),
        in_specs=[pl.BlockSpec((_RB, HID), lambda g: (g, 0)),
                  pl.BlockSpec((1, HID), lambda g: (0, 0)),
                  pl.BlockSpec((HID, HID), lambda g: (0, 0))],
        out_specs=[pl.BlockSpec((_RB, HID), lambda g: (g, 0)),
                   pl.BlockSpec((_RB, HID), lambda g: (g, 0))],
        out_shape=[jax.ShapeDtypeStruct((NP, HID), jnp.float32),
                   jax.ShapeDtypeStruct((NP, HID), jnp.float32)],
    )(p, bvec.reshape(1, HID), Wn)


def _final_body(p_ref, b_ref, x1_ref, x2_ref, xc_ref, sc_ref):
    g = pl.program_id(0)
    x3 = jnp.tanh(p_ref[...] + b_ref[...])
    row = lax.broadcasted_iota(jnp.int32, (_RB, D), 0) + g * _RB
    xc = jnp.concatenate(
        [x1_ref[...], x2_ref[...], x3, jnp.zeros((_RB, HID), jnp.float32)],
        axis=1)
    xc_ref[...] = jnp.where(row < N, xc, 0.0)
    e31 = (lax.broadcasted_iota(jnp.int32, (1, HID), 1) == HID - 1)
    sc_ref[pl.ds(g, 1), :] = lax.dot_general(
        e31.astype(jnp.float32), x3, (((1,), (1,)), ((), ())),
        precision=lax.Precision.HIGHEST)


def _tc_final(p, bvec, x1, x2):
    return pl.pallas_call(
        _final_body,
        grid=(_GB,),
        in_specs=[pl.BlockSpec((_RB, HID), lambda g: (g, 0)),
                  pl.BlockSpec((1, HID), lambda g: (0, 0)),
                  pl.BlockSpec((_RB, HID), lambda g: (g, 0)),
                  pl.BlockSpec((_RB, HID), lambda g: (g, 0))],
        out_specs=[pl.BlockSpec((_RB, D), lambda g: (g, 0)),
                   pl.BlockSpec((_GB, _RB), lambda g: (0, 0))],
        out_shape=[jax.ShapeDtypeStruct((NP, D), jnp.float32),
                   jax.ShapeDtypeStruct((_GB, _RB), jnp.float32)],
    )(p, bvec.reshape(1, HID), x1, x2)


def _routing_body(priors_ref, out_ref):
    priors = priors_ref[...]
    out = jnp.mean(priors, axis=0)
    lane = lax.broadcasted_iota(jnp.int32, (OD, NUM_CLASSES), 0) // 32
    cls = lax.broadcasted_iota(jnp.int32, (OD, NUM_CLASSES), 1)
    S = (lane == cls).astype(jnp.float32)

    def block_sum(v):
        return jax.lax.dot(v, S, precision=lax.Precision.HIGHEST)

    def expand(v):
        return jax.lax.dot(v, S.T, precision=lax.Precision.HIGHEST)

    for _ in range(NUM_ITER):
        nrm = jnp.sqrt(block_sum(out * out))
        no = out * expand(1.0 / (nrm + 1e-12))
        logits = jnp.einsum('ibd,dc->ibc', priors * no[None], S,
                            precision=lax.Precision.HIGHEST)
        m = jnp.max(logits, axis=0)
        p = jnp.exp(logits - m[None])
        probs = p / jnp.sum(p, axis=0)[None]
        pe = jnp.einsum('ibc,dc->ibd', probs, S,
                        precision=lax.Precision.HIGHEST)
        out = jnp.sum(pe * priors, axis=0)

    sq = block_sum(out * out)
    scale = (sq / (1.0 + sq)) / jnp.sqrt(sq + 1e-12)
    out = out * expand(scale)
    out_ref[...] = jnp.sqrt(block_sum(out * out))


def _routing(priors_ibod):
    return pl.pallas_call(
        _routing_body,
        out_shape=jax.ShapeDtypeStruct((B, NUM_CLASSES), jnp.float32),
    )(priors_ibod)


def kernel(x, edge_index, batch, W1, b1, W2, b2, W3, b3, Wc):
    src0 = edge_index[0]
    dst0 = edge_index[1]
    srcp = jnp.pad(src0, (0, EPAD - E)).reshape(NTILE, ECH, 128)
    dstp = jnp.pad(dst0, (0, EPAD - E)).reshape(NTILE, ECH, 128)
    srcf = jnp.pad(src0, (0, EFLAT - E))
    dstf = jnp.pad(dst0, (0, EFLAT - E))
    batchp = jnp.pad(batch, (0, NTILE * BCH * 128 - N),
                     constant_values=64).reshape(NTILE, BCH, 128)
    x_pad = jnp.pad(x, ((0, NP - N), (0, 0)))

    degp, cntp, SL, SD, cnts = _edgeprep(srcp, dstp, batchp, srcf, dstf)
    deg = degp[0, :, 0] + degp[1, :, 0] + 1.0
    dinv = jnp.where(deg > 0, 1.0 / jnp.sqrt(jnp.maximum(deg, 1e-12)), 0.0)

    h1 = _tc_first(x_pad, W1)
    p1 = _sc_apply(h1, SL, SD, cnts, dinv)
    x1, h2 = _tc_layer(p1, b1, W2)
    p2 = _sc_apply(h2, SL, SD, cnts, dinv)
    x2, h3 = _tc_layer(p2, b2, W3)
    p3 = _sc_apply(h3, SL, SD, cnts, dinv)
    xc_pad, scores2d = _tc_final(p3, b3, x1, x2)

    scores = scores2d.reshape(NP)[:N]
    smat = jnp.where(batch[None, :] == jnp.arange(B)[:, None],
                     scores[None, :], -jnp.inf)
    vals, idx = jax.lax.top_k(smat, K_POOL)
    pooled = xc_pad[idx]
    pooled = jnp.where(jnp.isfinite(vals)[..., None], pooled, 0.0)

    Wt = jnp.transpose(Wc, (1, 3, 0, 2)).reshape(K_POOL, 96, OD)
    Wt = jnp.pad(Wt, ((0, 0), (0, 32), (0, 0)))
    priors = jnp.einsum('bic,icd->ibd', pooled, Wt,
                        precision=lax.Precision.HIGHEST)
    return _routing(priors)


# speedup vs baseline: 8.5926x; 1.0920x over previous
import functools

import jax
import jax.numpy as jnp
from jax import lax
from jax.experimental import pallas as pl
from jax.experimental.pallas import tpu as pltpu
from jax.experimental.pallas import tpu_sc as plsc

N = 10000
NP = 10240
E = 320000
D = 128
B = 64
NUM_CLASSES = 10
K_POOL = 50
NUM_ITER = 3
HID = 32
OD = NUM_CLASSES * 32
DUMMY = 10016

NTILE = 32
ECH = 79
EPT = ECH * 128
EPAD = NTILE * EPT
BCH = 3
STRIPE = NP // 16

ROWS = 320
WIN = 8192
NWIN = 40
EFLAT = NWIN * WIN
CAP = 16384

_MESH = plsc.VectorSubcoreMesh(core_axis_name="c", subcore_axis_name="s",
                               num_cores=2, num_subcores=16)
_SC_PARAMS = pltpu.CompilerParams(use_tc_tiling_on_sc=False,
                                  needs_layout_passes=False)


def _fill2d(ref, rows, val):
    def body(r, _):
        ref[r, pl.ds(0, 16)] = jnp.full((16,), val, ref.dtype)
        if ref.shape[1] == 32:
            ref[r, pl.ds(16, 16)] = jnp.full((16,), val, ref.dtype)
        return 0
    lax.fori_loop(0, rows, body, 0)


def _fill1d(ref, n, val):
    def body(i, _):
        ref[pl.ds(i * 16, 16)] = jnp.full((16,), val, ref.dtype)
        return 0
    lax.fori_loop(0, n // 16, body, 0)


def _edgeprep_body(src_hbm, dst_hbm, batch_hbm, srcf_hbm, dstf_hbm,
                   degp_hbm, cntp_hbm, sl_hbm, sd_hbm, cnts_hbm,
                   srcv, dstv, bv, ones, zb, sw, dw, listS, listD, ct,
                   acc, cacc):
    c = lax.axis_index("c")
    s = lax.axis_index("s")
    w = c * 16 + s
    pltpu.sync_copy(src_hbm.at[w], srcv)
    pltpu.sync_copy(dst_hbm.at[w], dstv)
    pltpu.sync_copy(batch_hbm.at[w], bv)
    _fill2d(ones, 128, 1.0)
    _fill2d(zb, 128, 0.0)
    for i in range(STRIPE // 128):
        pltpu.sync_copy(zb, acc.at[pl.ds(s * STRIPE + i * 128, 128)])
    @pl.when(s == 0)
    def _():
        pltpu.sync_copy(zb, cacc)
    plsc.subcore_barrier()

    def fix(i, _):
        j = i // 8
        t = (i % 8) * 16
        vs = srcv[j, pl.ds(t, 16)]
        vd = dstv[j, pl.ds(t, 16)]
        dstv[j, pl.ds(t, 16)] = jnp.where(vs == vd, DUMMY, vd)
        return 0
    lax.fori_loop(0, ECH * 8, fix, 0)

    def scat(j, _):
        pltpu.sync_copy(ones, acc.at[dstv.at[j]], add=True)
        return 0
    lax.fori_loop(0, ECH, scat, 0)
    for j in range(BCH):
        pltpu.sync_copy(ones, cacc.at[bv.at[j]], add=True)

    _fill1d(listS, CAP, 0)
    _fill1d(listD, CAP, DUMMY)
    r0 = w * ROWS
    iota = lax.iota(jnp.int32, 16)

    def win(jw, cnt):
        pltpu.sync_copy(srcf_hbm.at[pl.ds(jw * WIN, WIN)], sw)
        pltpu.sync_copy(dstf_hbm.at[pl.ds(jw * WIN, WIN)], dw)

        def chunk(k, cnt2):
            vs = sw[pl.ds(k * 16, 16)]
            vd = dw[pl.ds(k * 16, 16)]
            mask = (vd >= r0) & (vd < r0 + ROWS) & (vs != vd)
            tot = plsc.all_reduce_population_count(mask)[0]
            @pl.when(tot > 0)
            def _():
                rank = plsc.cumsum(jnp.where(mask, 1, 0))
                pos = cnt2 + rank - 1
                plsc.store_scatter(listS, [pos], vs, mask=mask)
                plsc.store_scatter(listD, [pos], vd, mask=mask)
            return cnt2 + tot
        return lax.fori_loop(0, WIN // 16, chunk, cnt)
    cnt = lax.fori_loop(0, NWIN, win, 0)

    pltpu.sync_copy(listS, sl_hbm.at[w])
    pltpu.sync_copy(listD, sd_hbm.at[w])
    ct[...] = jnp.where(iota == 0, cnt, 0)
    pltpu.sync_copy(ct, cnts_hbm.at[w])

    plsc.subcore_barrier()
    pltpu.sync_copy(acc.at[pl.ds(s * STRIPE, STRIPE)],
                    degp_hbm.at[c, pl.ds(s * STRIPE, STRIPE)])
    @pl.when(s == 0)
    def _():
        pltpu.sync_copy(cacc, cntp_hbm.at[c])


_edgeprep = functools.partial(
    pl.kernel, mesh=_MESH, compiler_params=_SC_PARAMS,
    out_type=(jax.ShapeDtypeStruct((2, NP, 16), jnp.float32),
              jax.ShapeDtypeStruct((2, 128, 16), jnp.float32),
              jax.ShapeDtypeStruct((NTILE, CAP), jnp.int32),
              jax.ShapeDtypeStruct((NTILE, CAP), jnp.int32),
              jax.ShapeDtypeStruct((NTILE, 16), jnp.int32)),
    scratch_types=[pltpu.VMEM((ECH, 128), jnp.int32),
                   pltpu.VMEM((ECH, 128), jnp.int32),
                   pltpu.VMEM((BCH, 128), jnp.int32),
                   pltpu.VMEM((128, 16), jnp.float32),
                   pltpu.VMEM((128, 16), jnp.float32),
                   pltpu.VMEM((WIN,), jnp.int32),
                   pltpu.VMEM((WIN,), jnp.int32),
                   pltpu.VMEM((CAP,), jnp.int32),
                   pltpu.VMEM((CAP,), jnp.int32),
                   pltpu.VMEM((16,), jnp.int32),
                   pltpu.VMEM_SHARED((NP, 16), jnp.float32),
                   pltpu.VMEM_SHARED((128, 16), jnp.float32)],
)(_edgeprep_body)


def _apply_body(h_hbm, sl_hbm, sd_hbm, cnts_hbm, dinv_hbm, p_hbm,
                lS, lD, dv, hb, g0, g1, ct, acc, sem0, sem1):
    c = lax.axis_index("c")
    s = lax.axis_index("s")
    w = c * 16 + s
    r0 = w * ROWS
    pltpu.sync_copy(sl_hbm.at[w], lS)
    pltpu.sync_copy(sd_hbm.at[w], lD)
    pltpu.sync_copy(dinv_hbm, dv)
    pltpu.sync_copy(h_hbm.at[pl.ds(r0, ROWS)], hb)
    pltpu.sync_copy(cnts_hbm.at[w], ct)
    _fill2d(acc, ROWS + 16, 0.0)
    cnt = ct[...][0]
    T = (cnt + 127) // 128
    iota = lax.iota(jnp.int32, 16)
    bufs = (g0, g1)
    sems = (sem0, sem1)

    def start(j, b):
        pltpu.async_copy(h_hbm.at[lS.at[pl.ds(j * 128, 128)]], bufs[b],
                         sems[b])

    def wait(b):
        pltpu.make_async_copy(h_hbm.at[pl.ds(0, 128)], bufs[b],
                              sems[b]).wait()

    def process(j, b):
        buf = bufs[b]

        def group(g, _):
            base = j * 128 + g * 16
            sv = lS[pl.ds(base, 16)]
            vd = lD[pl.ds(base, 16)]
            coef = plsc.load_gather(dv, [sv]) * plsc.load_gather(dv, [vd])
            inr = (vd >= r0) & (vd < r0 + ROWS)
            dloc = jnp.where(inr, vd - r0, ROWS)
            for u in range(16):
                c_u = coef[u]
                dl_u = dloc[u]
                r = g * 16 + u
                plsc.addupdate(acc.at[dl_u, pl.ds(0, 16)],
                               buf[r, pl.ds(0, 16)] * c_u)
                plsc.addupdate(acc.at[dl_u, pl.ds(16, 16)],
                               buf[r, pl.ds(16, 16)] * c_u)
            return 0
        lax.fori_loop(0, 8, group, 0)

    @pl.when(T > 0)
    def _():
        start(0, 0)

    def outer(jj, _):
        for b in range(2):
            j = jj * 2 + b
            @pl.when(j < T)
            def _():
                wait(b)
                @pl.when(j + 1 < T)
                def _():
                    start(j + 1, 1 - b)
                process(j, b)
        return 0
    lax.fori_loop(0, (T + 1) // 2, outer, 0)

    def selfrow(i, _):
        d16 = dv[pl.ds(r0 + i * 16, 16)]
        coef = d16 * d16
        for u in range(16):
            r = i * 16 + u
            n = r0 + r
            c_u = coef[u]
            @pl.when(n < N)
            def _():
                plsc.addupdate(acc.at[r, pl.ds(0, 16)],
                               hb[r, pl.ds(0, 16)] * c_u)
                plsc.addupdate(acc.at[r, pl.ds(16, 16)],
                               hb[r, pl.ds(16, 16)] * c_u)
        return 0
    lax.fori_loop(0, ROWS // 16, selfrow, 0)

    pltpu.sync_copy(acc.at[pl.ds(0, ROWS)], p_hbm.at[pl.ds(r0, ROWS)])


_sc_apply = functools.partial(
    pl.kernel, mesh=_MESH, compiler_params=_SC_PARAMS,
    out_type=jax.ShapeDtypeStruct((NP, HID), jnp.float32),
    scratch_types=[pltpu.VMEM((CAP,), jnp.int32),
                   pltpu.VMEM((CAP,), jnp.int32),
                   pltpu.VMEM((NP,), jnp.float32),
                   pltpu.VMEM((ROWS, HID), jnp.float32),
                   pltpu.VMEM((128, HID), jnp.float32),
                   pltpu.VMEM((128, HID), jnp.float32),
                   pltpu.VMEM((16,), jnp.int32),
                   pltpu.VMEM((ROWS + 16, HID), jnp.float32),
                   pltpu.SemaphoreType.DMA,
                   pltpu.SemaphoreType.DMA],
)(_apply_body)


_GB = 8
_RB = NP // _GB


def _first_body(x_ref, w_ref, h_ref):
    h_ref[...] = jax.lax.dot(x_ref[...], w_ref[...])


def _tc_first(x_pad, W1):
    return pl.pallas_call(
        _first_body,
        grid=(_GB,),
        in_specs=[pl.BlockSpec((_RB, D), lambda g: (g, 0)),
                  pl.BlockSpec((D, HID), lambda g: (0, 0))],
        out_specs=pl.BlockSpec((_RB, HID), lambda g: (g, 0)),
        out_shape=jax.ShapeDtypeStruct((NP, HID), jnp.float32),
    )(x_pad, W1)


def _layer_body(p_ref, b_ref, w_ref, x_ref, hn_ref):
    a = jnp.tanh(p_ref[...] + b_ref[...])
    x_ref[...] = a
    hn_ref[...] = jax.lax.dot(a, w_ref[...])


def _tc_layer(p, bvec, Wn):
    return pl.pallas_call(
        _layer_body,
        grid=(_GB,),
        in_specs=[pl.BlockSpec((_RB, HID), lambda g: (g, 0)),
                  pl.BlockSpec((1, HID), lambda g: (0, 0)),
                  pl.BlockSpec((HID, HID), lambda g: (0, 0))],
        out_specs=[pl.BlockSpec((_RB, HID), lambda g: (g, 0)),
                   pl.BlockSpec((_RB, HID), lambda g: (g, 0))],
        out_shape=[jax.ShapeDtypeStruct((NP, HID), jnp.float32),
                   jax.ShapeDtypeStruct((NP, HID), jnp.float32)],
    )(p, bvec.reshape(1, HID), Wn)


def _final_body(p_ref, b_ref, x1_ref, x2_ref, xc_ref, sc_ref):
    g = pl.program_id(0)
    x3 = jnp.tanh(p_ref[...] + b_ref[...])
    row = lax.broadcasted_iota(jnp.int32, (_RB, D), 0) + g * _RB
    xc = jnp.concatenate(
        [x1_ref[...], x2_ref[...], x3, jnp.zeros((_RB, HID), jnp.float32)],
        axis=1)
    xc_ref[...] = jnp.where(row < N, xc, 0.0)
    e31 = (lax.broadcasted_iota(jnp.int32, (1, HID), 1) == HID - 1)
    sc_ref[pl.ds(g, 1), :] = lax.dot_general(
        e31.astype(jnp.float32), x3, (((1,), (1,)), ((), ())),
        precision=lax.Precision.HIGHEST)


def _tc_final(p, bvec, x1, x2):
    return pl.pallas_call(
        _final_body,
        grid=(_GB,),
        in_specs=[pl.BlockSpec((_RB, HID), lambda g: (g, 0)),
                  pl.BlockSpec((1, HID), lambda g: (0, 0)),
                  pl.BlockSpec((_RB, HID), lambda g: (g, 0)),
                  pl.BlockSpec((_RB, HID), lambda g: (g, 0))],
        out_specs=[pl.BlockSpec((_RB, D), lambda g: (g, 0)),
                   pl.BlockSpec((_GB, _RB), lambda g: (0, 0))],
        out_shape=[jax.ShapeDtypeStruct((NP, D), jnp.float32),
                   jax.ShapeDtypeStruct((_GB, _RB), jnp.float32)],
    )(p, bvec.reshape(1, HID), x1, x2)


def _routing_body(priors_ref, out_ref):
    priors = priors_ref[...]
    out = jnp.mean(priors, axis=0)
    lane = lax.broadcasted_iota(jnp.int32, (OD, NUM_CLASSES), 0) // 32
    cls = lax.broadcasted_iota(jnp.int32, (OD, NUM_CLASSES), 1)
    S = (lane == cls).astype(jnp.float32)

    def block_sum(v):
        return jax.lax.dot(v, S, precision=lax.Precision.HIGHEST)

    def expand(v):
        return jax.lax.dot(v, S.T, precision=lax.Precision.HIGHEST)

    for _ in range(NUM_ITER):
        nrm = jnp.sqrt(block_sum(out * out))
        no = out * expand(1.0 / (nrm + 1e-12))
        logits = jnp.einsum('ibd,dc->ibc', priors * no[None], S,
                            precision=lax.Precision.HIGHEST)
        m = jnp.max(logits, axis=0)
        p = jnp.exp(logits - m[None])
        probs = p / jnp.sum(p, axis=0)[None]
        pe = jnp.einsum('ibc,dc->ibd', probs, S,
                        precision=lax.Precision.HIGHEST)
        out = jnp.sum(pe * priors, axis=0)

    sq = block_sum(out * out)
    scale = (sq / (1.0 + sq)) / jnp.sqrt(sq + 1e-12)
    out = out * expand(scale)
    out_ref[...] = jnp.sqrt(block_sum(out * out))


def _routing(priors_ibod):
    return pl.pallas_call(
        _routing_body,
        out_shape=jax.ShapeDtypeStruct((B, NUM_CLASSES), jnp.float32),
    )(priors_ibod)


def kernel(x, edge_index, batch, W1, b1, W2, b2, W3, b3, Wc):
    src0 = edge_index[0]
    dst0 = edge_index[1]
    srcp = jnp.pad(src0, (0, EPAD - E)).reshape(NTILE, ECH, 128)
    dstp = jnp.pad(dst0, (0, EPAD - E)).reshape(NTILE, ECH, 128)
    srcf = jnp.pad(src0, (0, EFLAT - E))
    dstf = jnp.pad(dst0, (0, EFLAT - E))
    batchp = jnp.pad(batch, (0, NTILE * BCH * 128 - N),
                     constant_values=64).reshape(NTILE, BCH, 128)
    x_pad = jnp.pad(x, ((0, NP - N), (0, 0)))

    degp, cntp, SL, SD, cnts = _edgeprep(srcp, dstp, batchp, srcf, dstf)
    deg = degp[0, :, 0] + degp[1, :, 0] + 1.0
    dinv = jnp.where(deg > 0, 1.0 / jnp.sqrt(jnp.maximum(deg, 1e-12)), 0.0)

    h1 = _tc_first(x_pad, W1)
    p1 = _sc_apply(h1, SL, SD, cnts, dinv)
    x1, h2 = _tc_layer(p1, b1, W2)
    p2 = _sc_apply(h2, SL, SD, cnts, dinv)
    x2, h3 = _tc_layer(p2, b2, W3)
    p3 = _sc_apply(h3, SL, SD, cnts, dinv)
    xc_pad, scores2d = _tc_final(p3, b3, x1, x2)

    scores = scores2d.reshape(NP)[:N]
    smat = jnp.where(batch[None, :] == jnp.arange(B)[:, None],
                     scores[None, :], -jnp.inf)
    vals, idx = jax.lax.top_k(smat, K_POOL)
    pooled = xc_pad[idx]
    pooled = jnp.where(jnp.isfinite(vals)[..., None], pooled, 0.0)

    Wt = jnp.transpose(Wc, (1, 3, 0, 2)).reshape(K_POOL, 96, OD)
    Wt = jnp.pad(Wt, ((0, 0), (0, 32), (0, 0)))
    priors = jnp.einsum('bic,icd->ibd', pooled, Wt,
                        precision=lax.Precision.HIGHEST)
    return _routing(priors)
